# baseline (speedup 1.0000x reference)
import functools

import jax
import jax.numpy as jnp
from jax import lax
from jax.experimental import pallas as pl
from jax.experimental.pallas import tpu as pltpu
from jax.experimental.pallas import tpu_sc as plsc

_MESH = plsc.VectorSubcoreMesh(core_axis_name="c", subcore_axis_name="s")


@functools.partial(
    pl.kernel,
    mesh=_MESH,
    out_type=jax.ShapeDtypeStruct((4, 6), jnp.float32),
)
def _sc_copy(src_hbm, out_hbm):
    wid = lax.axis_index("s") * 2 + lax.axis_index("c")

    @pl.when(wid == 0)
    def _():
        pltpu.sync_copy(src_hbm, out_hbm)


def kernel(src, test_buffer):
    del test_buffer
    return _sc_copy(src)


# speedup vs baseline: 1.1105x; 1.1105x over previous
import functools

import jax
import jax.numpy as jnp
from jax import lax
from jax.experimental import pallas as pl
from jax.experimental.pallas import tpu as pltpu
from jax.experimental.pallas import tpu_sc as plsc

_MESH = plsc.ScalarSubcoreMesh(axis_name="c", num_cores=2)


@functools.partial(
    pl.kernel,
    mesh=_MESH,
    out_type=jax.ShapeDtypeStruct((4, 6), jnp.float32),
)
def _sc_copy(src_hbm, out_hbm):
    cid = lax.axis_index("c")

    @pl.when(cid == 0)
    def _():
        pltpu.sync_copy(src_hbm, out_hbm)


def kernel(src, test_buffer):
    del test_buffer
    return _sc_copy(src)


# speedup vs baseline: 17.8104x; 16.0381x over previous
import jax
import jax.numpy as jnp
from jax.experimental import pallas as pl
from jax.experimental.pallas import tpu as pltpu


def _copy_kernel(src_hbm, out_hbm, sem):
    copy = pltpu.make_async_copy(src_hbm, out_hbm, sem)
    copy.start()
    copy.wait()


def kernel(src, test_buffer):
    del test_buffer
    return pl.pallas_call(
        _copy_kernel,
        in_specs=[pl.BlockSpec(memory_space=pl.ANY)],
        out_specs=pl.BlockSpec(memory_space=pl.ANY),
        out_shape=jax.ShapeDtypeStruct((4, 6), jnp.float32),
        scratch_shapes=[pltpu.SemaphoreType.DMA],
    )(src)
